# baseline (speedup 1.0000x reference)
import jax
import jax.numpy as jnp
from jax import lax
from jax.experimental import pallas as pl
from jax.experimental.pallas import tpu as pltpu
from jax.experimental.pallas import tpu_sc as plsc

NC = 2
NS = 16
L = 16
NW = NC * NS
B = 16384
D = 32
N = 1000000
BPW = B // NW
NG = BPW // L
CH = 128
NCH = BPW // CH


def _sc_body(iso_hbm, rand_hbm, idx1_hbm, idx2_hbm, beta_hbm, out_hbm,
             idx1_v, idx2_v, z1_v, z2_v, r1_v, r2_v, beta_v, out_v,
             sem_z, sem_r):
    wid = lax.axis_index("s") * NC + lax.axis_index("c")

    pltpu.sync_copy(idx1_hbm.at[pl.ds(wid * BPW, BPW)], idx1_v)
    pltpu.sync_copy(idx2_hbm.at[pl.ds(wid * BPW, BPW)], idx2_v)
    pltpu.sync_copy(beta_hbm, beta_v)

    copies = []
    for j in range(NCH):
        sl = pl.ds(j * CH, CH)
        copies.append(pltpu.async_copy(
            iso_hbm.at[idx1_v.at[pl.ds(j * CH, CH)]], z1_v.at[sl], sem_z))
        copies.append(pltpu.async_copy(
            iso_hbm.at[idx2_v.at[pl.ds(j * CH, CH)]], z2_v.at[sl], sem_z))
        copies.append(pltpu.async_copy(
            rand_hbm.at[idx1_v.at[pl.ds(j * CH, CH)]], r1_v.at[sl], sem_r))
        copies.append(pltpu.async_copy(
            rand_hbm.at[idx2_v.at[pl.ds(j * CH, CH)]], r2_v.at[sl], sem_r))
    for c in copies:
        c.wait()

    beta_vec = beta_v[...]
    iota = lax.iota(jnp.int32, L)

    def group(g, carry):
        sl = pl.ds(g * L, L)
        rows = g * L + iota
        acc = jnp.zeros((L,), jnp.float32)
        for d in range(D):
            col = jnp.full((L,), d, jnp.int32)
            a = plsc.load_gather(z1_v, [rows, col])
            b = plsc.load_gather(z2_v, [rows, col])
            df = a - b
            acc = acc + df * df
        seed = jnp.int32(0x5F3759DF) - (plsc.bitcast(acc, jnp.int32) >> 1)
        y = plsc.bitcast(seed, jnp.float32)
        h = acc * jnp.float32(0.5)
        for _ in range(3):
            y = y * (jnp.float32(1.5) - h * y * y)
        dist = acc * y
        zeros = jnp.zeros((L,), jnp.int32)
        r1 = plsc.load_gather(r1_v, [rows, zeros])
        r2 = plsc.load_gather(r2_v, [rows, zeros])
        out_v[sl] = r1 + r2 - beta_vec * dist
        return carry

    lax.fori_loop(0, NG, group, 0)
    pltpu.sync_copy(out_v, out_hbm.at[pl.ds(wid * BPW, BPW)])


def kernel(protein1_idx, protein2_idx, iso_emb, rand_eff, beta_iso):
    rand16 = jnp.broadcast_to(rand_eff.astype(jnp.float32), (N, 16))
    beta = jnp.full((L,), beta_iso, jnp.float32)
    mesh = plsc.VectorSubcoreMesh(
        core_axis_name="c", subcore_axis_name="s",
        num_cores=NC, num_subcores=NS)
    run = pl.kernel(
        _sc_body,
        out_type=jax.ShapeDtypeStruct((B,), jnp.float32),
        mesh=mesh,
        compiler_params=pltpu.CompilerParams(
            needs_layout_passes=False, use_tc_tiling_on_sc=False),
        scratch_types=[
            pltpu.VMEM((BPW,), jnp.int32),
            pltpu.VMEM((BPW,), jnp.int32),
            pltpu.VMEM((BPW, D), jnp.float32),
            pltpu.VMEM((BPW, D), jnp.float32),
            pltpu.VMEM((BPW, 16), jnp.float32),
            pltpu.VMEM((BPW, 16), jnp.float32),
            pltpu.VMEM((L,), jnp.float32),
            pltpu.VMEM((BPW,), jnp.float32),
            pltpu.SemaphoreType.DMA,
            pltpu.SemaphoreType.DMA,
        ],
    )
    return run(iso_emb, rand16, protein1_idx, protein2_idx, beta)


# speedup vs baseline: 2.1303x; 2.1303x over previous
import jax
import jax.numpy as jnp
from jax import lax
from jax.experimental import pallas as pl
from jax.experimental.pallas import tpu as pltpu
from jax.experimental.pallas import tpu_sc as plsc

NC = 2
NS = 16
L = 16
NW = NC * NS
B = 16384
D = 32
N = 1000000
BPW = B // NW
NG = BPW // L
CH = 128
NCH = BPW // CH


def _sc_body(iso_hbm, rand_hbm, idx1_hbm, idx2_hbm, beta_hbm, out_hbm,
             idx1_v, idx2_v, z1_v, z2_v, r1_v, r2_v, beta_v, out_v,
             sem_z, sem_r):
    wid = lax.axis_index("s") * NC + lax.axis_index("c")

    pltpu.sync_copy(idx1_hbm.at[pl.ds(wid * BPW, BPW)], idx1_v)
    pltpu.sync_copy(idx2_hbm.at[pl.ds(wid * BPW, BPW)], idx2_v)
    pltpu.sync_copy(beta_hbm, beta_v)

    copies = []
    for j in range(NCH):
        sl = pl.ds(j * CH, CH)
        copies.append(pltpu.async_copy(
            iso_hbm.at[idx1_v.at[pl.ds(j * CH, CH)]], z1_v.at[sl], sem_z))
        copies.append(pltpu.async_copy(
            iso_hbm.at[idx2_v.at[pl.ds(j * CH, CH)]], z2_v.at[sl], sem_z))
        copies.append(pltpu.async_copy(
            rand_hbm.at[idx1_v.at[pl.ds(j * CH, CH)]], r1_v.at[sl], sem_r))
        copies.append(pltpu.async_copy(
            rand_hbm.at[idx2_v.at[pl.ds(j * CH, CH)]], r2_v.at[sl], sem_r))
    for c in copies:
        c.wait()

    beta_vec = beta_v[...]
    iota = lax.iota(jnp.int32, L)

    def group(g, carry):
        sl = pl.ds(g * L, L)
        rows = g * L + iota
        acc = jnp.zeros((L,), jnp.float32)
        for d in range(D):
            col = jnp.full((L,), d, jnp.int32)
            a = plsc.load_gather(z1_v, [rows, col])
            b = plsc.load_gather(z2_v, [rows, col])
            df = a - b
            acc = acc + df * df
        seed = jnp.int32(0x5F3759DF) - (plsc.bitcast(acc, jnp.int32) >> 1)
        y = plsc.bitcast(seed, jnp.float32)
        h = acc * jnp.float32(0.5)
        for _ in range(3):
            y = y * (jnp.float32(1.5) - h * y * y)
        dist = acc * y
        out_v[sl] = r1_v[sl] + r2_v[sl] - beta_vec * dist
        return carry

    lax.fori_loop(0, NG, group, 0)
    pltpu.sync_copy(out_v, out_hbm.at[pl.ds(wid * BPW, BPW)])


def kernel(protein1_idx, protein2_idx, iso_emb, rand_eff, beta_iso):
    rand_1d = jnp.sum(rand_eff.astype(jnp.float32), axis=1)
    beta = jnp.full((L,), beta_iso, jnp.float32)
    mesh = plsc.VectorSubcoreMesh(
        core_axis_name="c", subcore_axis_name="s",
        num_cores=NC, num_subcores=NS)
    run = pl.kernel(
        _sc_body,
        out_type=jax.ShapeDtypeStruct((B,), jnp.float32),
        mesh=mesh,
        compiler_params=pltpu.CompilerParams(
            needs_layout_passes=False, use_tc_tiling_on_sc=False),
        scratch_types=[
            pltpu.VMEM((BPW,), jnp.int32),
            pltpu.VMEM((BPW,), jnp.int32),
            pltpu.VMEM((BPW, D), jnp.float32),
            pltpu.VMEM((BPW, D), jnp.float32),
            pltpu.VMEM((BPW,), jnp.float32),
            pltpu.VMEM((BPW,), jnp.float32),
            pltpu.VMEM((L,), jnp.float32),
            pltpu.VMEM((BPW,), jnp.float32),
            pltpu.SemaphoreType.DMA,
            pltpu.SemaphoreType.DMA,
        ],
    )
    return run(iso_emb, rand_1d, protein1_idx, protein2_idx, beta)
